# Initial kernel scaffold; baseline (speedup 1.0000x reference)
import jax
import jax.numpy as jnp
from jax.experimental import pallas as pl


def kernel(x, edge_index, batch, W1, b1, W2, b2, Wl, bl):
    raise NotImplementedError("write your pallas kernel here")


# speedup vs baseline: 28.8433x; 28.8433x over previous
import functools

import jax
import jax.numpy as jnp
from jax import lax
from jax.experimental import pallas as pl
from jax.experimental.pallas import tpu as pltpu
from jax.experimental.pallas import tpu_sc as plsc

N = 100000
E = 1600000
G = 64
D_IN, D_HID, D_OUT = 32, 16, 5

NC, NS = 2, 16
NW = NC * NS
TL = 256
EP = 1605632
ROWS = EP // TL
PAD_IDX = N
CH = 5000
NCH = N // CH
ZCH = 200
DH = D_HID // 2

_mesh = plsc.VectorSubcoreMesh(core_axis_name="c", subcore_axis_name="s",
                               num_cores=NC, num_subcores=NS)
_sc_params = pltpu.CompilerParams(use_tc_tiling_on_sc=False)


DEG_RT = ROWS // NW
DEG_MB = 4
DEG_NMB = DEG_RT // DEG_MB


@functools.partial(
    pl.kernel,
    out_type=jax.ShapeDtypeStruct((NC, N, 8), jnp.float32),
    mesh=_mesh,
    compiler_params=_sc_params,
    scratch_types=[
        pltpu.VMEM((DEG_MB, TL), jnp.int32),
        pltpu.VMEM((TL, 8), jnp.float32),
        pltpu.VMEM((ZCH, 8), jnp.float32),
        pltpu.VMEM((2000, 8), jnp.float32),
        pltpu.VMEM_SHARED((N + 8, 8), jnp.float32),
        pltpu.SemaphoreType.DMA,
    ],
)
def _deg_kernel(dst_hbm, zeros_hbm, ones_hbm, out_hbm,
                didx, ones_v, zv, bb, acc, sem):
    c = lax.axis_index("c")
    s = lax.axis_index("s")
    pltpu.sync_copy(zeros_hbm, zv)
    pltpu.sync_copy(ones_hbm, ones_v)
    for j in range(2):
        k = s + j * NS

        @pl.when(k < NCH)
        def _():
            def zbody(m, carry):
                pltpu.sync_copy(zv, acc.at[pl.ds(k * CH + m * ZCH, ZCH)])
                return carry
            lax.fori_loop(0, CH // ZCH, zbody, 0)

    @pl.when(s == 0)
    def _():
        pltpu.sync_copy(zv.at[pl.ds(0, 8)], acc.at[pl.ds(N, 8)])

    plsc.subcore_barrier()
    base = (c * NS + s) * DEG_RT

    def body(i, carry):
        pltpu.sync_copy(dst_hbm.at[pl.ds(base + i * DEG_MB, DEG_MB)], didx)
        descs = [pltpu.async_copy(ones_v, acc.at[didx.at[j]], sem, add=True)
                 for j in range(DEG_MB)]
        for d in descs:
            d.wait()
        return carry

    lax.fori_loop(0, DEG_NMB, body, 0)
    plsc.subcore_barrier()
    for j in range(2):
        k = s + j * NS

        @pl.when(k < NCH)
        def _():
            for m in range(2):
                off = k * CH + m * 2000
                pltpu.sync_copy(acc.at[pl.ds(off, 2000)], bb)
                pltpu.sync_copy(bb, out_hbm.at[c, pl.ds(off, 2000)])
            off = k * CH + 4000
            pltpu.sync_copy(acc.at[pl.ds(off, 1000)], bb.at[pl.ds(0, 1000)])
            pltpu.sync_copy(bb.at[pl.ds(0, 1000)], out_hbm.at[c, pl.ds(off, 1000)])


AGG_RT = ROWS // NS
AGG_MB = 8
AGG_NMB = AGG_RT // AGG_MB


@functools.partial(
    pl.kernel,
    out_type=jax.ShapeDtypeStruct((NC, N, DH), jnp.float32),
    mesh=_mesh,
    compiler_params=_sc_params,
    scratch_types=[
        pltpu.VMEM((AGG_MB, TL), jnp.int32),
        pltpu.VMEM((AGG_MB, TL), jnp.int32),
        pltpu.VMEM((AGG_MB * TL, DH), jnp.float32),
        pltpu.VMEM((ZCH, DH), jnp.float32),
        pltpu.VMEM_SHARED((N + 8, DH), jnp.float32),
        pltpu.SemaphoreType.DMA,
        pltpu.SemaphoreType.DMA,
    ],
)
def _agg_kernel(src2_hbm, dst2_hbm, y_hbm, zeros_hbm, out_hbm,
                sidx, didx, rows_v, zb, acc, gsem, ssem):
    c = lax.axis_index("c")
    s = lax.axis_index("s")
    pltpu.sync_copy(zeros_hbm, zb)
    for j in range(2):
        k = s + j * NS

        @pl.when(k < NCH)
        def _():
            def zbody(m, carry):
                pltpu.sync_copy(zb, acc.at[pl.ds(k * CH + m * ZCH, ZCH)])
                return carry
            lax.fori_loop(0, CH // ZCH, zbody, 0)

    @pl.when(s == 0)
    def _():
        pltpu.sync_copy(zb.at[pl.ds(0, 8)], acc.at[pl.ds(N, 8)])

    plsc.subcore_barrier()
    base = s * AGG_RT

    def body(i, carry):
        r0 = base + i * AGG_MB
        pltpu.sync_copy(src2_hbm.at[c, pl.ds(r0, AGG_MB)], sidx)
        pltpu.sync_copy(dst2_hbm.at[pl.ds(r0, AGG_MB)], didx)
        gd = [pltpu.async_copy(y_hbm.at[sidx.at[j]],
                               rows_v.at[pl.ds(j * TL, TL)], gsem)
              for j in range(AGG_MB)]
        for d in gd:
            d.wait()
        sd = [pltpu.async_copy(rows_v.at[pl.ds(j * TL, TL)],
                               acc.at[didx.at[j]], ssem, add=True)
              for j in range(AGG_MB)]
        for d in sd:
            d.wait()
        return carry

    lax.fori_loop(0, AGG_NMB, body, 0)
    plsc.subcore_barrier()
    for j in range(2):
        k = s + j * NS

        @pl.when(k < NCH)
        def _():
            for m in range(2):
                off = k * CH + m * 2000
                pltpu.sync_copy(acc.at[pl.ds(off, 2000)],
                                rows_v.at[pl.ds(0, 2000)])
                pltpu.sync_copy(rows_v.at[pl.ds(0, 2000)],
                                out_hbm.at[c, pl.ds(off, 2000)])
            off = k * CH + 4000
            pltpu.sync_copy(acc.at[pl.ds(off, 1000)], rows_v.at[pl.ds(0, 1000)])
            pltpu.sync_copy(rows_v.at[pl.ds(0, 1000)],
                            out_hbm.at[c, pl.ds(off, 1000)])


BN = 5000
NB = N // BN


def _mm1_body(x_ref, d0_ref, d1_ref, w_ref, y_ref, dv_ref):
    deg = d0_ref[..., 0:1] + d1_ref[..., 0:1] + 1.0
    dinv = lax.rsqrt(deg)
    y_ref[...] = dinv * jnp.dot(x_ref[...], w_ref[...],
                                preferred_element_type=jnp.float32)
    dv_ref[...] = dinv


def _mm1_call(x, d0, d1, W1):
    return pl.pallas_call(
        _mm1_body,
        grid=(NB,),
        in_specs=[
            pl.BlockSpec((BN, D_IN), lambda i: (i, 0)),
            pl.BlockSpec((BN, 8), lambda i: (i, 0)),
            pl.BlockSpec((BN, 8), lambda i: (i, 0)),
            pl.BlockSpec((D_IN, D_HID), lambda i: (0, 0)),
        ],
        out_specs=[
            pl.BlockSpec((BN, D_HID), lambda i: (i, 0)),
            pl.BlockSpec((BN, 1), lambda i: (i, 0)),
        ],
        out_shape=[
            jax.ShapeDtypeStruct((N, D_HID), jnp.float32),
            jax.ShapeDtypeStruct((N, 1), jnp.float32),
        ],
    )(x, d0, d1, W1)


def _mid_body(a0_ref, a1_ref, y1_ref, dv_ref, b1_ref, w2_ref, y2_ref):
    dinv = dv_ref[...]
    agg = jnp.concatenate([a0_ref[...], a1_ref[...]], axis=1)
    h = dinv * (agg + y1_ref[...]) + b1_ref[...]
    h = jnp.maximum(h, 0.0)
    y2_ref[...] = dinv * jnp.dot(h, w2_ref[...],
                                 preferred_element_type=jnp.float32)


def _mid_call(a0, a1, y1, dv, b1, W2):
    return pl.pallas_call(
        _mid_body,
        grid=(NB,),
        in_specs=[
            pl.BlockSpec((BN, DH), lambda i: (i, 0)),
            pl.BlockSpec((BN, DH), lambda i: (i, 0)),
            pl.BlockSpec((BN, D_HID), lambda i: (i, 0)),
            pl.BlockSpec((BN, 1), lambda i: (i, 0)),
            pl.BlockSpec((1, D_HID), lambda i: (0, 0)),
            pl.BlockSpec((D_HID, D_HID), lambda i: (0, 0)),
        ],
        out_specs=pl.BlockSpec((BN, D_HID), lambda i: (i, 0)),
        out_shape=jax.ShapeDtypeStruct((N, D_HID), jnp.float32),
    )(a0, a1, y1, dv, b1, W2)


def _fin_body(a0_ref, a1_ref, y2_ref, dv_ref, b2_ref, bt_ref, wl_ref, bl_ref,
              out_ref, sums, cnt):
    i = pl.program_id(0)

    @pl.when(i == 0)
    def _():
        sums[...] = jnp.zeros_like(sums)
        cnt[...] = jnp.zeros_like(cnt)

    agg = jnp.concatenate([a0_ref[...], a1_ref[...]], axis=1)
    h2 = dv_ref[...] * (agg + y2_ref[...]) + b2_ref[...]
    oh = (bt_ref[...] == lax.broadcasted_iota(jnp.int32, (BN, G), 1))
    oh = oh.astype(jnp.float32)
    sums[...] += lax.dot_general(oh, h2, (((0,), (0,)), ((), ())),
                                 preferred_element_type=jnp.float32)
    cnt[...] += lax.dot_general(oh, jnp.ones((BN, 1), jnp.float32),
                                (((0,), (0,)), ((), ())),
                                preferred_element_type=jnp.float32)

    @pl.when(i == NB - 1)
    def _():
        pooled = sums[...] / jnp.maximum(cnt[...], 1.0)
        logits = jnp.dot(pooled, wl_ref[...],
                         preferred_element_type=jnp.float32) + bl_ref[...]
        m = jnp.max(logits, axis=1, keepdims=True)
        e = jnp.exp(logits - m)
        out_ref[...] = e / jnp.sum(e, axis=1, keepdims=True)


def _fin_call(a0, a1, y2, dv, b2, bt, Wl, bl):
    return pl.pallas_call(
        _fin_body,
        grid=(NB,),
        in_specs=[
            pl.BlockSpec((BN, DH), lambda i: (i, 0)),
            pl.BlockSpec((BN, DH), lambda i: (i, 0)),
            pl.BlockSpec((BN, D_HID), lambda i: (i, 0)),
            pl.BlockSpec((BN, 1), lambda i: (i, 0)),
            pl.BlockSpec((1, D_HID), lambda i: (0, 0)),
            pl.BlockSpec((BN, 1), lambda i: (i, 0)),
            pl.BlockSpec((D_HID, D_OUT), lambda i: (0, 0)),
            pl.BlockSpec((1, D_OUT), lambda i: (0, 0)),
        ],
        out_specs=pl.BlockSpec((G, D_OUT), lambda i: (0, 0)),
        out_shape=jax.ShapeDtypeStruct((G, D_OUT), jnp.float32),
        scratch_shapes=[
            pltpu.VMEM((G, D_HID), jnp.float32),
            pltpu.VMEM((G, 1), jnp.float32),
        ],
    )(a0, a1, y2, dv, b2, bt, Wl, bl)


def kernel(x, edge_index, batch, W1, b1, W2, b2, Wl, bl):
    ei = edge_index.astype(jnp.int32)
    src = ei[0]
    dst = ei[1]
    pad = EP - E
    srcp = jnp.concatenate([src, jnp.zeros((pad,), jnp.int32)])
    dstp = jnp.concatenate([dst, jnp.full((pad,), PAD_IDX, jnp.int32)])
    src2 = jnp.stack([2 * srcp, 2 * srcp + 1]).reshape(2, ROWS, TL)
    dst2 = dstp.reshape(ROWS, TL)
    zeros8_c = jnp.zeros((ZCH, DH), jnp.float32)
    ones_c = jnp.ones((TL, 8), jnp.float32)

    deg2 = _deg_kernel(dst2, zeros8_c, ones_c)
    y1, dinv = _mm1_call(x, deg2[0], deg2[1], W1)
    agg1 = _agg_kernel(src2, dst2, y1.reshape(2 * N, DH), zeros8_c)
    y2 = _mid_call(agg1[0], agg1[1], y1, dinv,
                   b1.reshape(1, D_HID), W2)
    agg2 = _agg_kernel(src2, dst2, y2.reshape(2 * N, DH), zeros8_c)
    return _fin_call(agg2[0], agg2[1], y2, dinv,
                     b2.reshape(1, D_HID), batch.astype(jnp.int32).reshape(N, 1),
                     Wl, bl.reshape(1, D_OUT))


# speedup vs baseline: 31.7884x; 1.1021x over previous
import functools

import jax
import jax.numpy as jnp
from jax import lax
from jax.experimental import pallas as pl
from jax.experimental.pallas import tpu as pltpu
from jax.experimental.pallas import tpu_sc as plsc

N = 100000
E = 1600000
G = 64
D_IN, D_HID, D_OUT = 32, 16, 5

NC, NS = 2, 16
NW = NC * NS
TL = 256
EP = 1605632
ROWS = EP // TL
PAD_IDX = N
CH = 5000
NCH = N // CH
ZCH = 200
DH = D_HID // 2

_mesh = plsc.VectorSubcoreMesh(core_axis_name="c", subcore_axis_name="s",
                               num_cores=NC, num_subcores=NS)
_sc_params = pltpu.CompilerParams(use_tc_tiling_on_sc=False)


DEG_RT = ROWS // NW
DEG_MB = 14
DEG_NMB = DEG_RT // DEG_MB


@functools.partial(
    pl.kernel,
    out_type=jax.ShapeDtypeStruct((NC, N, 8), jnp.float32),
    mesh=_mesh,
    compiler_params=_sc_params,
    scratch_types=[
        pltpu.VMEM((DEG_MB, TL), jnp.int32),
        pltpu.VMEM((TL, 8), jnp.float32),
        pltpu.VMEM((ZCH, 8), jnp.float32),
        pltpu.VMEM((2000, 8), jnp.float32),
        pltpu.VMEM_SHARED((N + 8, 8), jnp.float32),
        pltpu.SemaphoreType.DMA,
    ],
)
def _deg_kernel(dst_hbm, zeros_hbm, ones_hbm, out_hbm,
                didx, ones_v, zv, bb, acc, sem):
    c = lax.axis_index("c")
    s = lax.axis_index("s")
    pltpu.sync_copy(zeros_hbm, zv)
    pltpu.sync_copy(ones_hbm, ones_v)
    for j in range(2):
        k = s + j * NS

        @pl.when(k < NCH)
        def _():
            def zbody(m, carry):
                pltpu.sync_copy(zv, acc.at[pl.ds(k * CH + m * ZCH, ZCH)])
                return carry
            lax.fori_loop(0, CH // ZCH, zbody, 0)

    @pl.when(s == 0)
    def _():
        pltpu.sync_copy(zv.at[pl.ds(0, 8)], acc.at[pl.ds(N, 8)])

    plsc.subcore_barrier()
    base = (c * NS + s) * DEG_RT

    def body(i, carry):
        pltpu.sync_copy(dst_hbm.at[pl.ds(base + i * DEG_MB, DEG_MB)], didx)
        descs = [pltpu.async_copy(ones_v, acc.at[didx.at[j]], sem, add=True)
                 for j in range(DEG_MB)]
        for d in descs:
            d.wait()
        return carry

    lax.fori_loop(0, DEG_NMB, body, 0)
    plsc.subcore_barrier()
    for j in range(2):
        k = s + j * NS

        @pl.when(k < NCH)
        def _():
            for m in range(2):
                off = k * CH + m * 2000
                pltpu.sync_copy(acc.at[pl.ds(off, 2000)], bb)
                pltpu.sync_copy(bb, out_hbm.at[c, pl.ds(off, 2000)])
            off = k * CH + 4000
            pltpu.sync_copy(acc.at[pl.ds(off, 1000)], bb.at[pl.ds(0, 1000)])
            pltpu.sync_copy(bb.at[pl.ds(0, 1000)], out_hbm.at[c, pl.ds(off, 1000)])


AGG_RT = ROWS // NS
AGG_MB = 14
AGG_NMB = AGG_RT // AGG_MB


@functools.partial(
    pl.kernel,
    out_type=jax.ShapeDtypeStruct((NC, N, DH), jnp.float32),
    mesh=_mesh,
    compiler_params=_sc_params,
    scratch_types=[
        pltpu.VMEM((AGG_MB, TL), jnp.int32),
        pltpu.VMEM((AGG_MB, TL), jnp.int32),
        pltpu.VMEM((AGG_MB * TL, DH), jnp.float32),
        pltpu.VMEM((ZCH, DH), jnp.float32),
        pltpu.VMEM_SHARED((N + 8, DH), jnp.float32),
        pltpu.SemaphoreType.DMA,
        pltpu.SemaphoreType.DMA,
    ],
)
def _agg_kernel(src2_hbm, dst2_hbm, y_hbm, zeros_hbm, out_hbm,
                sidx, didx, rows_v, zb, acc, gsem, ssem):
    c = lax.axis_index("c")
    s = lax.axis_index("s")
    pltpu.sync_copy(zeros_hbm, zb)
    for j in range(2):
        k = s + j * NS

        @pl.when(k < NCH)
        def _():
            def zbody(m, carry):
                pltpu.sync_copy(zb, acc.at[pl.ds(k * CH + m * ZCH, ZCH)])
                return carry
            lax.fori_loop(0, CH // ZCH, zbody, 0)

    @pl.when(s == 0)
    def _():
        pltpu.sync_copy(zb.at[pl.ds(0, 8)], acc.at[pl.ds(N, 8)])

    plsc.subcore_barrier()
    base = s * AGG_RT

    def body(i, carry):
        r0 = base + i * AGG_MB
        pltpu.sync_copy(src2_hbm.at[c, pl.ds(r0, AGG_MB)], sidx)
        pltpu.sync_copy(dst2_hbm.at[pl.ds(r0, AGG_MB)], didx)
        gd = [pltpu.async_copy(y_hbm.at[sidx.at[j]],
                               rows_v.at[pl.ds(j * TL, TL)], gsem)
              for j in range(AGG_MB)]
        sd = []
        for j in range(AGG_MB):
            gd[j].wait()
            sd.append(pltpu.async_copy(rows_v.at[pl.ds(j * TL, TL)],
                                       acc.at[didx.at[j]], ssem, add=True))
        for d in sd:
            d.wait()
        return carry

    lax.fori_loop(0, AGG_NMB, body, 0)
    plsc.subcore_barrier()
    for j in range(2):
        k = s + j * NS

        @pl.when(k < NCH)
        def _():
            for m in range(2):
                off = k * CH + m * 2000
                pltpu.sync_copy(acc.at[pl.ds(off, 2000)],
                                rows_v.at[pl.ds(0, 2000)])
                pltpu.sync_copy(rows_v.at[pl.ds(0, 2000)],
                                out_hbm.at[c, pl.ds(off, 2000)])
            off = k * CH + 4000
            pltpu.sync_copy(acc.at[pl.ds(off, 1000)], rows_v.at[pl.ds(0, 1000)])
            pltpu.sync_copy(rows_v.at[pl.ds(0, 1000)],
                            out_hbm.at[c, pl.ds(off, 1000)])


BN = 5000
NB = N // BN


def _mm1_body(x_ref, d0_ref, d1_ref, w_ref, y_ref, dv_ref):
    deg = d0_ref[..., 0:1] + d1_ref[..., 0:1] + 1.0
    dinv = lax.rsqrt(deg)
    y_ref[...] = dinv * jnp.dot(x_ref[...], w_ref[...],
                                preferred_element_type=jnp.float32)
    dv_ref[...] = dinv


def _mm1_call(x, d0, d1, W1):
    return pl.pallas_call(
        _mm1_body,
        grid=(NB,),
        in_specs=[
            pl.BlockSpec((BN, D_IN), lambda i: (i, 0)),
            pl.BlockSpec((BN, 8), lambda i: (i, 0)),
            pl.BlockSpec((BN, 8), lambda i: (i, 0)),
            pl.BlockSpec((D_IN, D_HID), lambda i: (0, 0)),
        ],
        out_specs=[
            pl.BlockSpec((BN, D_HID), lambda i: (i, 0)),
            pl.BlockSpec((BN, 1), lambda i: (i, 0)),
        ],
        out_shape=[
            jax.ShapeDtypeStruct((N, D_HID), jnp.float32),
            jax.ShapeDtypeStruct((N, 1), jnp.float32),
        ],
    )(x, d0, d1, W1)


def _mid_body(a0_ref, a1_ref, y1_ref, dv_ref, b1_ref, w2_ref, y2_ref):
    dinv = dv_ref[...]
    agg = jnp.concatenate([a0_ref[...], a1_ref[...]], axis=1)
    h = dinv * (agg + y1_ref[...]) + b1_ref[...]
    h = jnp.maximum(h, 0.0)
    y2_ref[...] = dinv * jnp.dot(h, w2_ref[...],
                                 preferred_element_type=jnp.float32)


def _mid_call(a0, a1, y1, dv, b1, W2):
    return pl.pallas_call(
        _mid_body,
        grid=(NB,),
        in_specs=[
            pl.BlockSpec((BN, DH), lambda i: (i, 0)),
            pl.BlockSpec((BN, DH), lambda i: (i, 0)),
            pl.BlockSpec((BN, D_HID), lambda i: (i, 0)),
            pl.BlockSpec((BN, 1), lambda i: (i, 0)),
            pl.BlockSpec((1, D_HID), lambda i: (0, 0)),
            pl.BlockSpec((D_HID, D_HID), lambda i: (0, 0)),
        ],
        out_specs=pl.BlockSpec((BN, D_HID), lambda i: (i, 0)),
        out_shape=jax.ShapeDtypeStruct((N, D_HID), jnp.float32),
    )(a0, a1, y1, dv, b1, W2)


def _fin_body(a0_ref, a1_ref, y2_ref, dv_ref, b2_ref, bt_ref, wl_ref, bl_ref,
              out_ref, sums, cnt):
    i = pl.program_id(0)

    @pl.when(i == 0)
    def _():
        sums[...] = jnp.zeros_like(sums)
        cnt[...] = jnp.zeros_like(cnt)

    agg = jnp.concatenate([a0_ref[...], a1_ref[...]], axis=1)
    h2 = dv_ref[...] * (agg + y2_ref[...]) + b2_ref[...]
    oh = (bt_ref[...] == lax.broadcasted_iota(jnp.int32, (BN, G), 1))
    oh = oh.astype(jnp.float32)
    sums[...] += lax.dot_general(oh, h2, (((0,), (0,)), ((), ())),
                                 preferred_element_type=jnp.float32)
    cnt[...] += lax.dot_general(oh, jnp.ones((BN, 1), jnp.float32),
                                (((0,), (0,)), ((), ())),
                                preferred_element_type=jnp.float32)

    @pl.when(i == NB - 1)
    def _():
        pooled = sums[...] / jnp.maximum(cnt[...], 1.0)
        logits = jnp.dot(pooled, wl_ref[...],
                         preferred_element_type=jnp.float32) + bl_ref[...]
        m = jnp.max(logits, axis=1, keepdims=True)
        e = jnp.exp(logits - m)
        out_ref[...] = e / jnp.sum(e, axis=1, keepdims=True)


def _fin_call(a0, a1, y2, dv, b2, bt, Wl, bl):
    return pl.pallas_call(
        _fin_body,
        grid=(NB,),
        in_specs=[
            pl.BlockSpec((BN, DH), lambda i: (i, 0)),
            pl.BlockSpec((BN, DH), lambda i: (i, 0)),
            pl.BlockSpec((BN, D_HID), lambda i: (i, 0)),
            pl.BlockSpec((BN, 1), lambda i: (i, 0)),
            pl.BlockSpec((1, D_HID), lambda i: (0, 0)),
            pl.BlockSpec((BN, 1), lambda i: (i, 0)),
            pl.BlockSpec((D_HID, D_OUT), lambda i: (0, 0)),
            pl.BlockSpec((1, D_OUT), lambda i: (0, 0)),
        ],
        out_specs=pl.BlockSpec((G, D_OUT), lambda i: (0, 0)),
        out_shape=jax.ShapeDtypeStruct((G, D_OUT), jnp.float32),
        scratch_shapes=[
            pltpu.VMEM((G, D_HID), jnp.float32),
            pltpu.VMEM((G, 1), jnp.float32),
        ],
    )(a0, a1, y2, dv, b2, bt, Wl, bl)


def kernel(x, edge_index, batch, W1, b1, W2, b2, Wl, bl):
    ei = edge_index.astype(jnp.int32)
    src = ei[0]
    dst = ei[1]
    pad = EP - E
    srcp = jnp.concatenate([src, jnp.zeros((pad,), jnp.int32)])
    dstp = jnp.concatenate([dst, jnp.full((pad,), PAD_IDX, jnp.int32)])
    src2 = jnp.stack([2 * srcp, 2 * srcp + 1]).reshape(2, ROWS, TL)
    dst2 = dstp.reshape(ROWS, TL)
    zeros8_c = jnp.zeros((ZCH, DH), jnp.float32)
    ones_c = jnp.ones((TL, 8), jnp.float32)

    deg2 = _deg_kernel(dst2, zeros8_c, ones_c)
    y1, dinv = _mm1_call(x, deg2[0], deg2[1], W1)
    agg1 = _agg_kernel(src2, dst2, y1.reshape(2 * N, DH), zeros8_c)
    y2 = _mid_call(agg1[0], agg1[1], y1, dinv,
                   b1.reshape(1, D_HID), W2)
    agg2 = _agg_kernel(src2, dst2, y2.reshape(2 * N, DH), zeros8_c)
    return _fin_call(agg2[0], agg2[1], y2, dinv,
                     b2.reshape(1, D_HID), batch.astype(jnp.int32).reshape(N, 1),
                     Wl, bl.reshape(1, D_OUT))
